# Initial kernel scaffold; baseline (speedup 1.0000x reference)
import jax
import jax.numpy as jnp
from jax.experimental import pallas as pl


def kernel(x, W_mlp, W_router, W1, W2, W3):
    raise NotImplementedError("write your pallas kernel here")


# speedup vs baseline: 3.3927x; 3.3927x over previous
import jax
import jax.numpy as jnp
from jax.experimental import pallas as pl
from jax.experimental.pallas import tpu as pltpu

_D = 16
_H = 32
_E = 8
_TILE = 2048


def _dot(a, b, dims):
    return jax.lax.dot_general(a, b, (dims, ((), ())),
                               preferred_element_type=jnp.float32)


def _body(x_ref, wm_ref, wr_ref, w1_ref, w3_ref, w2_ref, rm_ref, o_ref):
    x = x_ref[...]
    t = _dot(x, wm_ref[...], ((1,), (1,)))
    logits = _dot(t, wr_ref[...], ((1,), (1,)))

    m = jnp.max(logits, axis=-1, keepdims=True)
    ex = jnp.exp(logits - m)
    s = ex / jnp.sum(ex, axis=-1, keepdims=True)

    ids = jax.lax.broadcasted_iota(jnp.int32, s.shape, 1)
    v1 = jnp.max(s, axis=-1, keepdims=True)
    i1 = jnp.min(jnp.where(s == v1, ids, _E), axis=-1, keepdims=True)
    m1 = ids == i1
    s2 = jnp.where(m1, -jnp.inf, s)
    v2 = jnp.max(s2, axis=-1, keepdims=True)
    i2 = jnp.min(jnp.where(s2 == v2, ids, _E), axis=-1, keepdims=True)
    m2 = ids == i2
    w = s * (m1.astype(s.dtype) + m2.astype(s.dtype))

    h1 = _dot(t, w1_ref[...], ((1,), (1,)))
    h3 = _dot(t, w3_ref[...], ((1,), (1,)))
    g = (h1 * jax.nn.sigmoid(h1)) * h3

    wrep = _dot(w, rm_ref[...], ((1,), (0,)))
    o_ref[...] = _dot(g * wrep, w2_ref[...], ((1,), (0,)))


def kernel(x, W_mlp, W_router, W1, W2, W3):
    B, S, D = x.shape
    T = B * S
    xt = x.reshape(T, D)
    w1f = W1.reshape(_E * _H, D)
    w3f = W3.reshape(_E * _H, D)
    w2f = W2.transpose(0, 2, 1).reshape(_E * _H, D)
    rm = jnp.repeat(jnp.eye(_E, dtype=jnp.float32), _H, axis=1)

    grid = (T // _TILE,)
    out = pl.pallas_call(
        _body,
        grid=grid,
        in_specs=[
            pl.BlockSpec((_TILE, D), lambda i: (i, 0)),
            pl.BlockSpec((_D, _D), lambda i: (0, 0)),
            pl.BlockSpec((_E, _D), lambda i: (0, 0)),
            pl.BlockSpec((_E * _H, _D), lambda i: (0, 0)),
            pl.BlockSpec((_E * _H, _D), lambda i: (0, 0)),
            pl.BlockSpec((_E * _H, _D), lambda i: (0, 0)),
            pl.BlockSpec((_E, _E * _H), lambda i: (0, 0)),
        ],
        out_specs=pl.BlockSpec((_TILE, D), lambda i: (i, 0)),
        out_shape=jax.ShapeDtypeStruct((T, D), jnp.float32),
        compiler_params=pltpu.CompilerParams(
            dimension_semantics=("arbitrary",),
        ),
    )(xt, W_mlp, W_router, w1f, w3f, w2f, rm)
    return out.reshape(B, S, D)


# speedup vs baseline: 3.5923x; 1.0588x over previous
import jax
import jax.numpy as jnp
from jax.experimental import pallas as pl
from jax.experimental.pallas import tpu as pltpu

_D = 16
_H = 32
_E = 8
_P = 8
_EH = _E * _H
_TR = 512


def _dot(a, b):
    return jax.lax.dot_general(a, b, ((((1,), (0,)), ((), ()))),
                               preferred_element_type=jnp.float32)


def _body(x_ref, ar_ref, ss_ref, zc_ref, a1_ref, a3_ref, rm_ref, w2_ref,
          o_ref):
    x = x_ref[...]
    logits = _dot(x, ar_ref[...])

    c = jnp.max(logits, axis=-1, keepdims=True)
    ex = jnp.exp(logits - c)
    s = ex / _dot(ex, ss_ref[...])

    sall = _dot(s, zc_ref[...])
    rank = jnp.zeros_like(s)
    one = jnp.ones_like(s)
    zero = jnp.zeros_like(s)
    for k in range(7):
        f = sall[:, (2 * k) * 128:(2 * k) * 128 + 64]
        b = sall[:, (2 * k + 1) * 128:(2 * k + 1) * 128 + 64]
        rank += jnp.where(f > s, one, zero)
        rank += jnp.where(b >= s, one, zero)
    w = jnp.where(rank < 1.5, s, zero)

    h1 = _dot(x, a1_ref[...])
    h3 = _dot(x, a3_ref[...])
    g = (h1 * jax.nn.sigmoid(h1)) * h3
    wrep = _dot(w, rm_ref[...])
    o_ref[...] = _dot(g * wrep, w2_ref[...])


def kernel(x, W_mlp, W_router, W1, W2, W3):
    B, S, D = x.shape
    T = B * S
    R = T // _P
    xp = x.reshape(R, _P * D)

    eyeP = jnp.eye(_P, dtype=jnp.float32)
    kron = jnp.kron

    ar = kron(eyeP, (W_router @ W_mlp).T)
    a1 = kron(eyeP, (W1.reshape(_EH, D) @ W_mlp).T)
    a3 = kron(eyeP, (W3.reshape(_EH, D) @ W_mlp).T)
    w2 = kron(eyeP, W2.transpose(0, 2, 1).reshape(_EH, D))
    ss = kron(eyeP, jnp.ones((_E, _E), jnp.float32))
    rm = kron(eyeP, jnp.repeat(jnp.eye(_E, dtype=jnp.float32), _H, axis=1))

    pad = jnp.zeros((_P * _E, 64), jnp.float32)
    blocks = []
    for r in range(1, _E):
        blocks.append(kron(eyeP, jnp.eye(_E, k=-r, dtype=jnp.float32)))
        blocks.append(pad)
        blocks.append(kron(eyeP, jnp.eye(_E, k=r, dtype=jnp.float32)))
        blocks.append(pad)
    zc = jnp.concatenate(blocks, axis=1)

    grid = (R // _TR,)
    out = pl.pallas_call(
        _body,
        grid=grid,
        in_specs=[
            pl.BlockSpec((_TR, _P * D), lambda i: (i, 0)),
            pl.BlockSpec(ar.shape, lambda i: (0, 0)),
            pl.BlockSpec(ss.shape, lambda i: (0, 0)),
            pl.BlockSpec(zc.shape, lambda i: (0, 0)),
            pl.BlockSpec(a1.shape, lambda i: (0, 0)),
            pl.BlockSpec(a3.shape, lambda i: (0, 0)),
            pl.BlockSpec(rm.shape, lambda i: (0, 0)),
            pl.BlockSpec(w2.shape, lambda i: (0, 0)),
        ],
        out_specs=pl.BlockSpec((_TR, _P * D), lambda i: (i, 0)),
        out_shape=jax.ShapeDtypeStruct((R, _P * D), jnp.float32),
        compiler_params=pltpu.CompilerParams(
            dimension_semantics=("arbitrary",),
        ),
    )(xp, ar, ss, zc, a1, a3, rm, w2)
    return out.reshape(B, S, D)
